# baseline (speedup 1.0000x reference)
import functools

import jax
import jax.numpy as jnp
from jax import lax
from jax.experimental import pallas as pl
from jax.experimental.pallas import tpu as pltpu
from jax.experimental.pallas import tpu_sc as plsc

N = 10000
NPAD = 10240
E = 320000
EPAD = 327680
NCORES = 2
NSUB = 16
NTILES = NCORES * NSUB
CHUNK = 128
CHUNKS_PER_TILE = EPAD // (NTILES * CHUNK)
ROWS_PER_TILE = NPAD // NSUB
ZROWS = 16
DEGW = 128
F = 128
NROW_BLK = 1024
GRID = NPAD // NROW_BLK


def _vec_mesh():
    return plsc.VectorSubcoreMesh(core_axis_name="c", subcore_axis_name="s")


def _make_sc_degree(width):
    @functools.partial(
        pl.kernel,
        mesh=_vec_mesh(),
        out_type=jax.ShapeDtypeStruct((NCORES * NPAD, width), jnp.float32),
        scratch_types=[
            pltpu.VMEM_SHARED((NPAD, width), jnp.float32),
            pltpu.VMEM((CHUNK,), jnp.int32),
            pltpu.VMEM((CHUNK, width), jnp.float32),
            pltpu.VMEM((ZROWS, width), jnp.float32),
        ],
    )
    def _sc_degree(row_hbm, out_hbm, acc, idx_v, ones_v, zbuf):
        c = lax.axis_index("c")
        s = lax.axis_index("s")
        wid = c * NSUB + s
        zero = jnp.zeros((16,), jnp.float32)
        one = jnp.ones((16,), jnp.float32)

        @pl.loop(0, ZROWS)
        def _(i):
            for j in range(width // 16):
                zbuf[i, pl.ds(j * 16, 16)] = zero

        @pl.loop(0, CHUNK)
        def _(i):
            for j in range(width // 16):
                ones_v[i, pl.ds(j * 16, 16)] = one

        @pl.loop(0, ROWS_PER_TILE // ZROWS)
        def _(i):
            pltpu.sync_copy(zbuf, acc.at[pl.ds(s * ROWS_PER_TILE + i * ZROWS, ZROWS)])

        plsc.subcore_barrier()

        base0 = wid * (CHUNKS_PER_TILE * CHUNK)

        @pl.loop(0, CHUNKS_PER_TILE)
        def _(j):
            pltpu.sync_copy(row_hbm.at[pl.ds(base0 + j * CHUNK, CHUNK)], idx_v)
            pltpu.sync_copy(ones_v, acc.at[idx_v], add=True)

        plsc.subcore_barrier()
        pltpu.sync_copy(
            acc.at[pl.ds(s * ROWS_PER_TILE, ROWS_PER_TILE)],
            out_hbm.at[pl.ds(c * NPAD + s * ROWS_PER_TILE, ROWS_PER_TILE)],
        )

    return _sc_degree


_sc_degree = _make_sc_degree(DEGW)


@functools.partial(
    pl.kernel,
    mesh=_vec_mesh(),
    out_type=jax.ShapeDtypeStruct((NCORES * NPAD, F), jnp.float32),
    scratch_types=[
        pltpu.VMEM_SHARED((NPAD, F), jnp.float32),
        pltpu.VMEM((CHUNKS_PER_TILE, CHUNK), jnp.int32),
        pltpu.VMEM((CHUNK, F), jnp.float32),
        pltpu.VMEM((CHUNK, F), jnp.float32),
        pltpu.VMEM((CHUNK,), jnp.int32),
        pltpu.VMEM((CHUNK,), jnp.int32),
        pltpu.SemaphoreType.DMA,
        pltpu.SemaphoreType.DMA,
        pltpu.SemaphoreType.DMA,
        pltpu.SemaphoreType.DMA,
    ],
)
def _sc_scatter(h_hbm, row_hbm, col_hbm, out_hbm, acc, ridx,
                rows_v0, rows_v1, cbuf0, cbuf1, sem0, sem1, semc0, semc1):
    c = lax.axis_index("c")
    s = lax.axis_index("s")
    wid = c * NSUB + s
    zero = jnp.zeros((16,), jnp.float32)
    cbase = wid * CHUNKS_PER_TILE

    cp_r = pltpu.make_async_copy(
        row_hbm.at[pl.ds(cbase, CHUNKS_PER_TILE)], ridx, sem0)
    cp_r.start()

    @pl.loop(0, CHUNK)
    def _(i):
        for j in range(F // 16):
            rows_v0[i, pl.ds(j * 16, 16)] = zero

    @pl.loop(0, ROWS_PER_TILE // CHUNK)
    def _(i):
        pltpu.sync_copy(rows_v0, acc.at[pl.ds(s * ROWS_PER_TILE + i * CHUNK, CHUNK)])

    cp_r.wait()
    plsc.subcore_barrier()

    pltpu.make_async_copy(h_hbm.at[ridx.at[0]], rows_v0, sem0).start()
    pltpu.make_async_copy(col_hbm.at[cbase], cbuf0, semc0).start()

    @pl.loop(0, CHUNKS_PER_TILE // 2)
    def _(k):
        j = 2 * k
        pltpu.make_async_copy(h_hbm.at[ridx.at[j + 1]], rows_v1, sem1).start()
        pltpu.make_async_copy(col_hbm.at[cbase + j + 1], cbuf1, semc1).start()
        pltpu.make_async_copy(h_hbm.at[ridx.at[j]], rows_v0, sem0).wait()
        pltpu.make_async_copy(col_hbm.at[cbase + j], cbuf0, semc0).wait()
        pltpu.sync_copy(rows_v0, acc.at[cbuf0], add=True)

        @pl.when(k < CHUNKS_PER_TILE // 2 - 1)
        def _():
            pltpu.make_async_copy(h_hbm.at[ridx.at[j + 2]], rows_v0, sem0).start()
            pltpu.make_async_copy(col_hbm.at[cbase + j + 2], cbuf0, semc0).start()

        pltpu.make_async_copy(h_hbm.at[ridx.at[j + 1]], rows_v1, sem1).wait()
        pltpu.make_async_copy(col_hbm.at[cbase + j + 1], cbuf1, semc1).wait()
        pltpu.sync_copy(rows_v1, acc.at[cbuf1], add=True)

    plsc.subcore_barrier()
    pltpu.sync_copy(
        acc.at[pl.ds(s * ROWS_PER_TILE, ROWS_PER_TILE)],
        out_hbm.at[pl.ds(c * NPAD + s * ROWS_PER_TILE, ROWS_PER_TILE)],
    )


def _lin1_body(x_ref, f_ref, w_ref, b_ref, o_ref):
    h = jnp.concatenate([x_ref[...], f_ref[...]], axis=-1)
    o_ref[...] = (
        jnp.dot(h, w_ref[...], preferred_element_type=jnp.float32) + b_ref[...]
    )


def _tc_lin1(x_p, feat_p, W1, b1):
    return pl.pallas_call(
        _lin1_body,
        grid=(GRID,),
        in_specs=[
            pl.BlockSpec((NROW_BLK, 64), lambda i: (i, 0)),
            pl.BlockSpec((NROW_BLK, 64), lambda i: (i, 0)),
            pl.BlockSpec((F, F), lambda i: (0, 0)),
            pl.BlockSpec((1, F), lambda i: (0, 0)),
        ],
        out_specs=pl.BlockSpec((NROW_BLK, F), lambda i: (i, 0)),
        out_shape=jax.ShapeDtypeStruct((NPAD, F), jnp.float32),
    )(x_p, feat_p, W1, b1.reshape(1, F))


def _mk_hs_body(d0_ref, d1_ref, lin_ref, dis_ref, hs_ref):
    i = pl.program_id(0)
    deg = d0_ref[...][:, :1] + d1_ref[...][:, :1] + 1.0
    rowid = i * NROW_BLK + lax.broadcasted_iota(jnp.int32, (NROW_BLK, 1), 0)
    dis = jnp.where(rowid < N, lax.rsqrt(deg), 0.0)
    dis_ref[...] = dis
    hs_ref[...] = dis * lin_ref[...]


def _tc_mk_hs(d0, d1, lin1):
    return pl.pallas_call(
        _mk_hs_body,
        grid=(GRID,),
        in_specs=[
            pl.BlockSpec((NROW_BLK, DEGW), lambda i: (i, 0)),
            pl.BlockSpec((NROW_BLK, DEGW), lambda i: (i, 0)),
            pl.BlockSpec((NROW_BLK, F), lambda i: (i, 0)),
        ],
        out_specs=[
            pl.BlockSpec((NROW_BLK, 1), lambda i: (i, 0)),
            pl.BlockSpec((NROW_BLK, F), lambda i: (i, 0)),
        ],
        out_shape=[
            jax.ShapeDtypeStruct((NPAD, 1), jnp.float32),
            jax.ShapeDtypeStruct((NPAD, F), jnp.float32),
        ],
    )(d0, d1, lin1)


def _combine_body(a0_ref, a1_ref, hs_ref, dis_ref, w_ref, b_ref, o_ref):
    dis = dis_ref[...]
    out1 = jnp.maximum(dis * (a0_ref[...] + a1_ref[...] + hs_ref[...]), 0.0)
    o_ref[...] = dis * (
        jnp.dot(out1, w_ref[...], preferred_element_type=jnp.float32) + b_ref[...]
    )


def _tc_combine(a0, a1, hs1, dis, W2, b2):
    return pl.pallas_call(
        _combine_body,
        grid=(GRID,),
        in_specs=[
            pl.BlockSpec((NROW_BLK, F), lambda i: (i, 0)),
            pl.BlockSpec((NROW_BLK, F), lambda i: (i, 0)),
            pl.BlockSpec((NROW_BLK, F), lambda i: (i, 0)),
            pl.BlockSpec((NROW_BLK, 1), lambda i: (i, 0)),
            pl.BlockSpec((F, F), lambda i: (0, 0)),
            pl.BlockSpec((1, F), lambda i: (0, 0)),
        ],
        out_specs=pl.BlockSpec((NROW_BLK, F), lambda i: (i, 0)),
        out_shape=jax.ShapeDtypeStruct((NPAD, F), jnp.float32),
    )(a0, a1, hs1, dis, W2, b2.reshape(1, F))


def _final_body(a0_ref, a1_ref, hs_ref, dis_ref, wp_ref, bp_ref, wq_ref, bq_ref,
                finx_ref, soft_ref, finy_ref, g_ref):
    i = pl.program_id(0)
    dis = dis_ref[...]
    out2 = jnp.maximum(dis * (a0_ref[...] + a1_ref[...] + hs_ref[...]), 0.0)
    m = jnp.max(out2, axis=0, keepdims=True)

    @pl.when(i == 0)
    def _():
        g_ref[...] = m

    @pl.when(i > 0)
    def _():
        g_ref[...] = jnp.maximum(g_ref[...], m)

    @pl.when(i == GRID - 1)
    def _():
        g = g_ref[...]
        finx = jnp.dot(g, wp_ref[...], preferred_element_type=jnp.float32) + bp_ref[...]
        finx_ref[...] = finx
        mx = jnp.max(finx, axis=-1, keepdims=True)
        e = jnp.exp(finx - mx)
        soft_ref[...] = e / jnp.sum(e, axis=-1, keepdims=True)
        finy_ref[...] = (
            jnp.dot(g, wq_ref[...], preferred_element_type=jnp.float32) + bq_ref[...]
        )


def _tc_final(a0, a1, hs2, dis, Wp, bp, Wq, bq):
    np1 = Wp.shape[1]
    return pl.pallas_call(
        _final_body,
        grid=(GRID,),
        in_specs=[
            pl.BlockSpec((NROW_BLK, F), lambda i: (i, 0)),
            pl.BlockSpec((NROW_BLK, F), lambda i: (i, 0)),
            pl.BlockSpec((NROW_BLK, F), lambda i: (i, 0)),
            pl.BlockSpec((NROW_BLK, 1), lambda i: (i, 0)),
            pl.BlockSpec((F, np1), lambda i: (0, 0)),
            pl.BlockSpec((1, np1), lambda i: (0, 0)),
            pl.BlockSpec((F, 1), lambda i: (0, 0)),
            pl.BlockSpec((1, 1), lambda i: (0, 0)),
        ],
        out_specs=[
            pl.BlockSpec((1, np1), lambda i: (0, 0)),
            pl.BlockSpec((1, np1), lambda i: (0, 0)),
            pl.BlockSpec((1, 1), lambda i: (0, 0)),
        ],
        out_shape=[
            jax.ShapeDtypeStruct((1, np1), jnp.float32),
            jax.ShapeDtypeStruct((1, np1), jnp.float32),
            jax.ShapeDtypeStruct((1, 1), jnp.float32),
        ],
        scratch_shapes=[pltpu.VMEM((1, F), jnp.float32)],
    )(a0, a1, hs2, dis, Wp, bp.reshape(1, np1), Wq, bq.reshape(1, 1))


def kernel(x, feat, edge_index, W1, b1, W2, b2, Wp, bp, Wq, bq):
    row = edge_index[0]
    col = edge_index[1]
    pad = jnp.full((EPAD - E,), N, dtype=row.dtype)
    row_p = jnp.concatenate([row, pad])
    col_p = jnp.concatenate([col, pad])

    x_p = jnp.pad(x, ((0, NPAD - N), (0, 0)))
    feat_p = jnp.pad(feat, ((0, NPAD - N), (0, 0)))

    row_off = jnp.concatenate([row_p[: EPAD // 2], row_p[EPAD // 2 :] + NPAD])
    row2 = row_off.reshape(NTILES * CHUNKS_PER_TILE, CHUNK)
    col2 = col_p.reshape(NTILES * CHUNKS_PER_TILE, CHUNK)

    degp = _sc_degree(row_p)
    lin1 = _tc_lin1(x_p, feat_p, W1, b1)
    dis, hs1 = _tc_mk_hs(degp[:NPAD], degp[NPAD:], lin1)

    accs1 = _sc_scatter(jnp.concatenate([hs1, hs1]), row2, col2)
    hs2 = _tc_combine(accs1[:NPAD], accs1[NPAD:], hs1, dis, W2, b2)

    accs2 = _sc_scatter(jnp.concatenate([hs2, hs2]), row2, col2)
    finx, soft, finy = _tc_final(accs2[:NPAD], accs2[NPAD:], hs2, dis, Wp, bp, Wq, bq)

    return finx.reshape(-1), soft.reshape(-1), finy.reshape(-1)


# speedup vs baseline: 1.0549x; 1.0549x over previous
import functools

import jax
import jax.numpy as jnp
from jax import lax
from jax.experimental import pallas as pl
from jax.experimental.pallas import tpu as pltpu
from jax.experimental.pallas import tpu_sc as plsc

N = 10000
NPAD = 10240
E = 320000
EPAD = 327680
NCORES = 2
NSUB = 16
NTILES = NCORES * NSUB
CHUNK = 128
CHUNKS_PER_TILE = EPAD // (NTILES * CHUNK)
NCHUNKS = EPAD // CHUNK
CH0 = 120
CH1 = 40
NCHUNKS_PAD = 2688
ROWS_PER_TILE = NPAD // NSUB
ZROWS = 16
DEGW = 128
F = 128
NROW_BLK = 1024
GRID = NPAD // NROW_BLK


def _vec_mesh():
    return plsc.VectorSubcoreMesh(core_axis_name="c", subcore_axis_name="s")


def _make_sc_degree(width):
    @functools.partial(
        pl.kernel,
        mesh=_vec_mesh(),
        out_type=jax.ShapeDtypeStruct((NCORES * NPAD, width), jnp.float32),
        scratch_types=[
            pltpu.VMEM_SHARED((NPAD, width), jnp.float32),
            pltpu.VMEM((CHUNK,), jnp.int32),
            pltpu.VMEM((CHUNK, width), jnp.float32),
            pltpu.VMEM((ZROWS, width), jnp.float32),
        ],
    )
    def _sc_degree(row_hbm, out_hbm, acc, idx_v, ones_v, zbuf):
        c = lax.axis_index("c")
        s = lax.axis_index("s")
        wid = c * NSUB + s
        zero = jnp.zeros((16,), jnp.float32)
        one = jnp.ones((16,), jnp.float32)

        @pl.loop(0, ZROWS)
        def _(i):
            for j in range(width // 16):
                zbuf[i, pl.ds(j * 16, 16)] = zero

        @pl.loop(0, CHUNK)
        def _(i):
            for j in range(width // 16):
                ones_v[i, pl.ds(j * 16, 16)] = one

        @pl.loop(0, ROWS_PER_TILE // ZROWS)
        def _(i):
            pltpu.sync_copy(zbuf, acc.at[pl.ds(s * ROWS_PER_TILE + i * ZROWS, ZROWS)])

        plsc.subcore_barrier()

        base0 = wid * (CHUNKS_PER_TILE * CHUNK)

        @pl.loop(0, CHUNKS_PER_TILE)
        def _(j):
            pltpu.sync_copy(row_hbm.at[pl.ds(base0 + j * CHUNK, CHUNK)], idx_v)
            pltpu.sync_copy(ones_v, acc.at[idx_v], add=True)

        plsc.subcore_barrier()
        pltpu.sync_copy(
            acc.at[pl.ds(s * ROWS_PER_TILE, ROWS_PER_TILE)],
            out_hbm.at[pl.ds(c * NPAD + s * ROWS_PER_TILE, ROWS_PER_TILE)],
        )

    return _sc_degree


_sc_degree = _make_sc_degree(DEGW)


@functools.partial(
    pl.kernel,
    mesh=_vec_mesh(),
    out_type=jax.ShapeDtypeStruct((NCORES * NPAD, F), jnp.float32),
    scratch_types=[
        pltpu.VMEM_SHARED((NPAD, F), jnp.float32),
        pltpu.VMEM((CH0, CHUNK), jnp.int32),
        pltpu.VMEM((CHUNK, F), jnp.float32),
        pltpu.VMEM((CHUNK, F), jnp.float32),
        pltpu.VMEM((CHUNK,), jnp.int32),
        pltpu.VMEM((CHUNK,), jnp.int32),
        pltpu.SemaphoreType.DMA,
        pltpu.SemaphoreType.DMA,
        pltpu.SemaphoreType.DMA,
        pltpu.SemaphoreType.DMA,
    ],
)
def _sc_scatter(h_hbm, row_hbm, col_hbm, out_hbm, acc, ridx,
                rows_v0, rows_v1, cbuf0, cbuf1, sem0, sem1, semc0, semc1):
    c = lax.axis_index("c")
    s = lax.axis_index("s")
    zero = jnp.zeros((16,), jnp.float32)
    cbase = jnp.where(c == 0, s * CH0, NSUB * CH0 + s * CH1)
    half = jnp.where(c == 0, CH0 // 2, CH1 // 2)

    cp_r = pltpu.make_async_copy(row_hbm.at[pl.ds(cbase, CH0)], ridx, sem0)
    cp_r.start()

    @pl.loop(0, CHUNK)
    def _(i):
        for j in range(F // 16):
            rows_v0[i, pl.ds(j * 16, 16)] = zero

    @pl.loop(0, ROWS_PER_TILE // CHUNK)
    def _(i):
        pltpu.sync_copy(rows_v0, acc.at[pl.ds(s * ROWS_PER_TILE + i * CHUNK, CHUNK)])

    cp_r.wait()
    plsc.subcore_barrier()

    pltpu.make_async_copy(h_hbm.at[ridx.at[0]], rows_v0, sem0).start()
    pltpu.make_async_copy(col_hbm.at[cbase], cbuf0, semc0).start()

    @pl.loop(0, CH0 // 2)
    def _(k):
        @pl.when(k < half)
        def _():
            j = 2 * k
            pltpu.make_async_copy(h_hbm.at[ridx.at[j + 1]], rows_v1, sem1).start()
            pltpu.make_async_copy(col_hbm.at[cbase + j + 1], cbuf1, semc1).start()
            pltpu.make_async_copy(h_hbm.at[ridx.at[j]], rows_v0, sem0).wait()
            pltpu.make_async_copy(col_hbm.at[cbase + j], cbuf0, semc0).wait()
            pltpu.sync_copy(rows_v0, acc.at[cbuf0], add=True)

            @pl.when(k < half - 1)
            def _():
                pltpu.make_async_copy(h_hbm.at[ridx.at[j + 2]], rows_v0, sem0).start()
                pltpu.make_async_copy(col_hbm.at[cbase + j + 2], cbuf0, semc0).start()

            pltpu.make_async_copy(h_hbm.at[ridx.at[j + 1]], rows_v1, sem1).wait()
            pltpu.make_async_copy(col_hbm.at[cbase + j + 1], cbuf1, semc1).wait()
            pltpu.sync_copy(rows_v1, acc.at[cbuf1], add=True)

    plsc.subcore_barrier()
    pltpu.sync_copy(
        acc.at[pl.ds(s * ROWS_PER_TILE, ROWS_PER_TILE)],
        out_hbm.at[pl.ds(c * NPAD + s * ROWS_PER_TILE, ROWS_PER_TILE)],
    )


def _lin1_body(x_ref, f_ref, w_ref, b_ref, o_ref):
    h = jnp.concatenate([x_ref[...], f_ref[...]], axis=-1)
    o_ref[...] = (
        jnp.dot(h, w_ref[...], preferred_element_type=jnp.float32) + b_ref[...]
    )


def _tc_lin1(x_p, feat_p, W1, b1):
    return pl.pallas_call(
        _lin1_body,
        grid=(GRID,),
        in_specs=[
            pl.BlockSpec((NROW_BLK, 64), lambda i: (i, 0)),
            pl.BlockSpec((NROW_BLK, 64), lambda i: (i, 0)),
            pl.BlockSpec((F, F), lambda i: (0, 0)),
            pl.BlockSpec((1, F), lambda i: (0, 0)),
        ],
        out_specs=pl.BlockSpec((NROW_BLK, F), lambda i: (i, 0)),
        out_shape=jax.ShapeDtypeStruct((NPAD, F), jnp.float32),
    )(x_p, feat_p, W1, b1.reshape(1, F))


def _mk_hs_body(d0_ref, d1_ref, lin_ref, dis_ref, hs_ref):
    i = pl.program_id(0)
    deg = d0_ref[...][:, :1] + d1_ref[...][:, :1] + 1.0
    rowid = i * NROW_BLK + lax.broadcasted_iota(jnp.int32, (NROW_BLK, 1), 0)
    dis = jnp.where(rowid < N, lax.rsqrt(deg), 0.0)
    dis_ref[...] = dis
    hs_ref[...] = dis * lin_ref[...]


def _tc_mk_hs(d0, d1, lin1):
    return pl.pallas_call(
        _mk_hs_body,
        grid=(GRID,),
        in_specs=[
            pl.BlockSpec((NROW_BLK, DEGW), lambda i: (i, 0)),
            pl.BlockSpec((NROW_BLK, DEGW), lambda i: (i, 0)),
            pl.BlockSpec((NROW_BLK, F), lambda i: (i, 0)),
        ],
        out_specs=[
            pl.BlockSpec((NROW_BLK, 1), lambda i: (i, 0)),
            pl.BlockSpec((NROW_BLK, F), lambda i: (i, 0)),
        ],
        out_shape=[
            jax.ShapeDtypeStruct((NPAD, 1), jnp.float32),
            jax.ShapeDtypeStruct((NPAD, F), jnp.float32),
        ],
    )(d0, d1, lin1)


def _combine_body(a0_ref, a1_ref, hs_ref, dis_ref, w_ref, b_ref, o_ref):
    dis = dis_ref[...]
    out1 = jnp.maximum(dis * (a0_ref[...] + a1_ref[...] + hs_ref[...]), 0.0)
    o_ref[...] = dis * (
        jnp.dot(out1, w_ref[...], preferred_element_type=jnp.float32) + b_ref[...]
    )


def _tc_combine(a0, a1, hs1, dis, W2, b2):
    return pl.pallas_call(
        _combine_body,
        grid=(GRID,),
        in_specs=[
            pl.BlockSpec((NROW_BLK, F), lambda i: (i, 0)),
            pl.BlockSpec((NROW_BLK, F), lambda i: (i, 0)),
            pl.BlockSpec((NROW_BLK, F), lambda i: (i, 0)),
            pl.BlockSpec((NROW_BLK, 1), lambda i: (i, 0)),
            pl.BlockSpec((F, F), lambda i: (0, 0)),
            pl.BlockSpec((1, F), lambda i: (0, 0)),
        ],
        out_specs=pl.BlockSpec((NROW_BLK, F), lambda i: (i, 0)),
        out_shape=jax.ShapeDtypeStruct((NPAD, F), jnp.float32),
    )(a0, a1, hs1, dis, W2, b2.reshape(1, F))


def _final_body(a0_ref, a1_ref, hs_ref, dis_ref, wp_ref, bp_ref, wq_ref, bq_ref,
                finx_ref, soft_ref, finy_ref, g_ref):
    i = pl.program_id(0)
    dis = dis_ref[...]
    out2 = jnp.maximum(dis * (a0_ref[...] + a1_ref[...] + hs_ref[...]), 0.0)
    m = jnp.max(out2, axis=0, keepdims=True)

    @pl.when(i == 0)
    def _():
        g_ref[...] = m

    @pl.when(i > 0)
    def _():
        g_ref[...] = jnp.maximum(g_ref[...], m)

    @pl.when(i == GRID - 1)
    def _():
        g = g_ref[...]
        finx = jnp.dot(g, wp_ref[...], preferred_element_type=jnp.float32) + bp_ref[...]
        finx_ref[...] = finx
        mx = jnp.max(finx, axis=-1, keepdims=True)
        e = jnp.exp(finx - mx)
        soft_ref[...] = e / jnp.sum(e, axis=-1, keepdims=True)
        finy_ref[...] = (
            jnp.dot(g, wq_ref[...], preferred_element_type=jnp.float32) + bq_ref[...]
        )


def _tc_final(a0, a1, hs2, dis, Wp, bp, Wq, bq):
    np1 = Wp.shape[1]
    return pl.pallas_call(
        _final_body,
        grid=(GRID,),
        in_specs=[
            pl.BlockSpec((NROW_BLK, F), lambda i: (i, 0)),
            pl.BlockSpec((NROW_BLK, F), lambda i: (i, 0)),
            pl.BlockSpec((NROW_BLK, F), lambda i: (i, 0)),
            pl.BlockSpec((NROW_BLK, 1), lambda i: (i, 0)),
            pl.BlockSpec((F, np1), lambda i: (0, 0)),
            pl.BlockSpec((1, np1), lambda i: (0, 0)),
            pl.BlockSpec((F, 1), lambda i: (0, 0)),
            pl.BlockSpec((1, 1), lambda i: (0, 0)),
        ],
        out_specs=[
            pl.BlockSpec((1, np1), lambda i: (0, 0)),
            pl.BlockSpec((1, np1), lambda i: (0, 0)),
            pl.BlockSpec((1, 1), lambda i: (0, 0)),
        ],
        out_shape=[
            jax.ShapeDtypeStruct((1, np1), jnp.float32),
            jax.ShapeDtypeStruct((1, np1), jnp.float32),
            jax.ShapeDtypeStruct((1, 1), jnp.float32),
        ],
        scratch_shapes=[pltpu.VMEM((1, F), jnp.float32)],
    )(a0, a1, hs2, dis, Wp, bp.reshape(1, np1), Wq, bq.reshape(1, 1))


def kernel(x, feat, edge_index, W1, b1, W2, b2, Wp, bp, Wq, bq):
    row = edge_index[0]
    col = edge_index[1]
    pad = jnp.full((EPAD - E,), N, dtype=row.dtype)
    row_p = jnp.concatenate([row, pad])
    col_p = jnp.concatenate([col, pad])

    x_p = jnp.pad(x, ((0, NPAD - N), (0, 0)))
    feat_p = jnp.pad(feat, ((0, NPAD - N), (0, 0)))

    row2 = jnp.pad(row_p.reshape(NCHUNKS, CHUNK),
                   ((0, NCHUNKS_PAD - NCHUNKS), (0, 0)), constant_values=N)
    col2 = jnp.pad(col_p.reshape(NCHUNKS, CHUNK),
                   ((0, NCHUNKS_PAD - NCHUNKS), (0, 0)), constant_values=N)

    degp = _sc_degree(row_p)
    lin1 = _tc_lin1(x_p, feat_p, W1, b1)
    dis, hs1 = _tc_mk_hs(degp[:NPAD], degp[NPAD:], lin1)

    accs1 = _sc_scatter(hs1, row2, col2)
    hs2 = _tc_combine(accs1[:NPAD], accs1[NPAD:], hs1, dis, W2, b2)

    accs2 = _sc_scatter(hs2, row2, col2)
    finx, soft, finy = _tc_final(accs2[:NPAD], accs2[NPAD:], hs2, dis, Wp, bp, Wq, bq)

    return finx.reshape(-1), soft.reshape(-1), finy.reshape(-1)


# speedup vs baseline: 1.0551x; 1.0002x over previous
import functools

import jax
import jax.numpy as jnp
from jax import lax
from jax.experimental import pallas as pl
from jax.experimental.pallas import tpu as pltpu
from jax.experimental.pallas import tpu_sc as plsc

N = 10000
NPAD = 10240
E = 320000
EPAD = 327680
NCORES = 2
NSUB = 16
NTILES = NCORES * NSUB
CHUNK = 128
CHUNKS_PER_TILE = EPAD // (NTILES * CHUNK)
SCH = 128
NCHUNKS = EPAD // SCH
NSUB_G = 4
CH0 = 120
CH1 = 40
NCHUNKS_PAD = 2688
ROWS_PER_TILE = NPAD // NSUB
ZROWS = 16
DEGW = 128
F = 128
NROW_BLK = 1024
GRID = NPAD // NROW_BLK


def _vec_mesh():
    return plsc.VectorSubcoreMesh(core_axis_name="c", subcore_axis_name="s")


def _make_sc_degree(width):
    @functools.partial(
        pl.kernel,
        mesh=_vec_mesh(),
        out_type=jax.ShapeDtypeStruct((NCORES * NPAD, width), jnp.float32),
        scratch_types=[
            pltpu.VMEM_SHARED((NPAD, width), jnp.float32),
            pltpu.VMEM((CHUNK,), jnp.int32),
            pltpu.VMEM((CHUNK, width), jnp.float32),
            pltpu.VMEM((ZROWS, width), jnp.float32),
        ],
    )
    def _sc_degree(row_hbm, out_hbm, acc, idx_v, ones_v, zbuf):
        c = lax.axis_index("c")
        s = lax.axis_index("s")
        wid = c * NSUB + s
        zero = jnp.zeros((16,), jnp.float32)
        one = jnp.ones((16,), jnp.float32)

        @pl.loop(0, ZROWS)
        def _(i):
            for j in range(width // 16):
                zbuf[i, pl.ds(j * 16, 16)] = zero

        @pl.loop(0, CHUNK)
        def _(i):
            for j in range(width // 16):
                ones_v[i, pl.ds(j * 16, 16)] = one

        @pl.loop(0, ROWS_PER_TILE // ZROWS)
        def _(i):
            pltpu.sync_copy(zbuf, acc.at[pl.ds(s * ROWS_PER_TILE + i * ZROWS, ZROWS)])

        plsc.subcore_barrier()

        base0 = wid * (CHUNKS_PER_TILE * CHUNK)

        @pl.loop(0, CHUNKS_PER_TILE)
        def _(j):
            pltpu.sync_copy(row_hbm.at[pl.ds(base0 + j * CHUNK, CHUNK)], idx_v)
            pltpu.sync_copy(ones_v, acc.at[idx_v], add=True)

        plsc.subcore_barrier()
        pltpu.sync_copy(
            acc.at[pl.ds(s * ROWS_PER_TILE, ROWS_PER_TILE)],
            out_hbm.at[pl.ds(c * NPAD + s * ROWS_PER_TILE, ROWS_PER_TILE)],
        )

    return _sc_degree


_sc_degree = _make_sc_degree(DEGW)


@functools.partial(
    pl.kernel,
    mesh=_vec_mesh(),
    out_type=jax.ShapeDtypeStruct((NCORES * NPAD, F), jnp.float32),
    scratch_types=[
        pltpu.VMEM_SHARED((NPAD, F), jnp.float32),
        pltpu.VMEM((CH0, SCH), jnp.int32),
        pltpu.VMEM((2, SCH, F), jnp.float32),
        pltpu.VMEM((2, SCH), jnp.int32),
        pltpu.SemaphoreType.DMA,
        pltpu.SemaphoreType.DMA,
        pltpu.SemaphoreType.DMA,
        pltpu.SemaphoreType.DMA,
        pltpu.SemaphoreType.DMA,
        pltpu.SemaphoreType.DMA,
        pltpu.SemaphoreType.DMA,
        pltpu.SemaphoreType.DMA,
        pltpu.SemaphoreType.DMA,
        pltpu.SemaphoreType.DMA,
    ],
)
def _sc_scatter(h_hbm, row_hbm, col_hbm, out_hbm, acc, ridx, rows_v, cbuf,
                g00, g01, g02, g03, g10, g11, g12, g13, c0, c1):
    c = lax.axis_index("c")
    s = lax.axis_index("s")
    zero = jnp.zeros((16,), jnp.float32)
    cbase = jnp.where(c == 0, s * CH0, NSUB * CH0 + s * CH1)
    n = jnp.where(c == 0, CH0, CH1)
    gsems = ((g00, g01, g02, g03), (g10, g11, g12, g13))
    csems = (c0, c1)
    SUBR = SCH // NSUB_G

    cp_r = pltpu.make_async_copy(row_hbm.at[pl.ds(cbase, CH0)], ridx, g00)
    cp_r.start()

    @pl.loop(0, SCH)
    def _(i):
        for j in range(F // 16):
            rows_v[0, i, pl.ds(j * 16, 16)] = zero

    @pl.loop(0, ROWS_PER_TILE // SCH)
    def _(i):
        pltpu.sync_copy(rows_v.at[0], acc.at[pl.ds(s * ROWS_PER_TILE + i * SCH, SCH)])

    cp_r.wait()
    plsc.subcore_barrier()

    def start(j, b):
        for q in range(NSUB_G):
            pltpu.make_async_copy(
                h_hbm.at[ridx.at[j, pl.ds(q * SUBR, SUBR)]],
                rows_v.at[b, pl.ds(q * SUBR, SUBR)],
                gsems[b][q],
            ).start()
        pltpu.make_async_copy(col_hbm.at[cbase + j], cbuf.at[b], csems[b]).start()

    def finish(j, b):
        for q in range(NSUB_G):
            pltpu.make_async_copy(
                h_hbm.at[ridx.at[j, pl.ds(q * SUBR, SUBR)]],
                rows_v.at[b, pl.ds(q * SUBR, SUBR)],
                gsems[b][q],
            ).wait()
        pltpu.make_async_copy(col_hbm.at[cbase + j], cbuf.at[b], csems[b]).wait()
        pltpu.sync_copy(rows_v.at[b], acc.at[cbuf.at[b]], add=True)

    start(0, 0)

    @pl.loop(0, CH0 // 2)
    def _(k):
        @pl.when(k < n // 2)
        def _():
            j = 2 * k
            start(j + 1, 1)
            finish(j, 0)

            @pl.when(j + 2 < n)
            def _():
                start(j + 2, 0)

            finish(j + 1, 1)

    plsc.subcore_barrier()
    pltpu.sync_copy(
        acc.at[pl.ds(s * ROWS_PER_TILE, ROWS_PER_TILE)],
        out_hbm.at[pl.ds(c * NPAD + s * ROWS_PER_TILE, ROWS_PER_TILE)],
    )


def _lin1_body(x_ref, f_ref, w_ref, b_ref, o_ref):
    h = jnp.concatenate([x_ref[...], f_ref[...]], axis=-1)
    o_ref[...] = (
        jnp.dot(h, w_ref[...], preferred_element_type=jnp.float32) + b_ref[...]
    )


def _tc_lin1(x_p, feat_p, W1, b1):
    return pl.pallas_call(
        _lin1_body,
        grid=(GRID,),
        in_specs=[
            pl.BlockSpec((NROW_BLK, 64), lambda i: (i, 0)),
            pl.BlockSpec((NROW_BLK, 64), lambda i: (i, 0)),
            pl.BlockSpec((F, F), lambda i: (0, 0)),
            pl.BlockSpec((1, F), lambda i: (0, 0)),
        ],
        out_specs=pl.BlockSpec((NROW_BLK, F), lambda i: (i, 0)),
        out_shape=jax.ShapeDtypeStruct((NPAD, F), jnp.float32),
    )(x_p, feat_p, W1, b1.reshape(1, F))


def _mk_hs_body(d0_ref, d1_ref, lin_ref, dis_ref, hs_ref):
    i = pl.program_id(0)
    deg = d0_ref[...][:, :1] + d1_ref[...][:, :1] + 1.0
    rowid = i * NROW_BLK + lax.broadcasted_iota(jnp.int32, (NROW_BLK, 1), 0)
    dis = jnp.where(rowid < N, lax.rsqrt(deg), 0.0)
    dis_ref[...] = dis
    hs_ref[...] = dis * lin_ref[...]


def _tc_mk_hs(d0, d1, lin1):
    return pl.pallas_call(
        _mk_hs_body,
        grid=(GRID,),
        in_specs=[
            pl.BlockSpec((NROW_BLK, DEGW), lambda i: (i, 0)),
            pl.BlockSpec((NROW_BLK, DEGW), lambda i: (i, 0)),
            pl.BlockSpec((NROW_BLK, F), lambda i: (i, 0)),
        ],
        out_specs=[
            pl.BlockSpec((NROW_BLK, 1), lambda i: (i, 0)),
            pl.BlockSpec((NROW_BLK, F), lambda i: (i, 0)),
        ],
        out_shape=[
            jax.ShapeDtypeStruct((NPAD, 1), jnp.float32),
            jax.ShapeDtypeStruct((NPAD, F), jnp.float32),
        ],
    )(d0, d1, lin1)


def _combine_body(a0_ref, a1_ref, hs_ref, dis_ref, w_ref, b_ref, o_ref):
    dis = dis_ref[...]
    out1 = jnp.maximum(dis * (a0_ref[...] + a1_ref[...] + hs_ref[...]), 0.0)
    o_ref[...] = dis * (
        jnp.dot(out1, w_ref[...], preferred_element_type=jnp.float32) + b_ref[...]
    )


def _tc_combine(a0, a1, hs1, dis, W2, b2):
    return pl.pallas_call(
        _combine_body,
        grid=(GRID,),
        in_specs=[
            pl.BlockSpec((NROW_BLK, F), lambda i: (i, 0)),
            pl.BlockSpec((NROW_BLK, F), lambda i: (i, 0)),
            pl.BlockSpec((NROW_BLK, F), lambda i: (i, 0)),
            pl.BlockSpec((NROW_BLK, 1), lambda i: (i, 0)),
            pl.BlockSpec((F, F), lambda i: (0, 0)),
            pl.BlockSpec((1, F), lambda i: (0, 0)),
        ],
        out_specs=pl.BlockSpec((NROW_BLK, F), lambda i: (i, 0)),
        out_shape=jax.ShapeDtypeStruct((NPAD, F), jnp.float32),
    )(a0, a1, hs1, dis, W2, b2.reshape(1, F))


def _final_body(a0_ref, a1_ref, hs_ref, dis_ref, wp_ref, bp_ref, wq_ref, bq_ref,
                finx_ref, soft_ref, finy_ref, g_ref):
    i = pl.program_id(0)
    dis = dis_ref[...]
    out2 = jnp.maximum(dis * (a0_ref[...] + a1_ref[...] + hs_ref[...]), 0.0)
    m = jnp.max(out2, axis=0, keepdims=True)

    @pl.when(i == 0)
    def _():
        g_ref[...] = m

    @pl.when(i > 0)
    def _():
        g_ref[...] = jnp.maximum(g_ref[...], m)

    @pl.when(i == GRID - 1)
    def _():
        g = g_ref[...]
        finx = jnp.dot(g, wp_ref[...], preferred_element_type=jnp.float32) + bp_ref[...]
        finx_ref[...] = finx
        mx = jnp.max(finx, axis=-1, keepdims=True)
        e = jnp.exp(finx - mx)
        soft_ref[...] = e / jnp.sum(e, axis=-1, keepdims=True)
        finy_ref[...] = (
            jnp.dot(g, wq_ref[...], preferred_element_type=jnp.float32) + bq_ref[...]
        )


def _tc_final(a0, a1, hs2, dis, Wp, bp, Wq, bq):
    np1 = Wp.shape[1]
    return pl.pallas_call(
        _final_body,
        grid=(GRID,),
        in_specs=[
            pl.BlockSpec((NROW_BLK, F), lambda i: (i, 0)),
            pl.BlockSpec((NROW_BLK, F), lambda i: (i, 0)),
            pl.BlockSpec((NROW_BLK, F), lambda i: (i, 0)),
            pl.BlockSpec((NROW_BLK, 1), lambda i: (i, 0)),
            pl.BlockSpec((F, np1), lambda i: (0, 0)),
            pl.BlockSpec((1, np1), lambda i: (0, 0)),
            pl.BlockSpec((F, 1), lambda i: (0, 0)),
            pl.BlockSpec((1, 1), lambda i: (0, 0)),
        ],
        out_specs=[
            pl.BlockSpec((1, np1), lambda i: (0, 0)),
            pl.BlockSpec((1, np1), lambda i: (0, 0)),
            pl.BlockSpec((1, 1), lambda i: (0, 0)),
        ],
        out_shape=[
            jax.ShapeDtypeStruct((1, np1), jnp.float32),
            jax.ShapeDtypeStruct((1, np1), jnp.float32),
            jax.ShapeDtypeStruct((1, 1), jnp.float32),
        ],
        scratch_shapes=[pltpu.VMEM((1, F), jnp.float32)],
    )(a0, a1, hs2, dis, Wp, bp.reshape(1, np1), Wq, bq.reshape(1, 1))


def kernel(x, feat, edge_index, W1, b1, W2, b2, Wp, bp, Wq, bq):
    row = edge_index[0]
    col = edge_index[1]
    pad = jnp.full((EPAD - E,), N, dtype=row.dtype)
    row_p = jnp.concatenate([row, pad])
    col_p = jnp.concatenate([col, pad])

    x_p = jnp.pad(x, ((0, NPAD - N), (0, 0)))
    feat_p = jnp.pad(feat, ((0, NPAD - N), (0, 0)))

    row2 = jnp.pad(row_p.reshape(NCHUNKS, SCH),
                   ((0, NCHUNKS_PAD - NCHUNKS), (0, 0)), constant_values=N)
    col2 = jnp.pad(col_p.reshape(NCHUNKS, SCH),
                   ((0, NCHUNKS_PAD - NCHUNKS), (0, 0)), constant_values=N)

    degp = _sc_degree(row_p)
    lin1 = _tc_lin1(x_p, feat_p, W1, b1)
    dis, hs1 = _tc_mk_hs(degp[:NPAD], degp[NPAD:], lin1)

    accs1 = _sc_scatter(hs1, row2, col2)
    hs2 = _tc_combine(accs1[:NPAD], accs1[NPAD:], hs1, dis, W2, b2)

    accs2 = _sc_scatter(hs2, row2, col2)
    finx, soft, finy = _tc_final(accs2[:NPAD], accs2[NPAD:], hs2, dis, Wp, bp, Wq, bq)

    return finx.reshape(-1), soft.reshape(-1), finy.reshape(-1)
